# baseline (speedup 1.0000x reference)
import jax
import jax.numpy as jnp
from jax.experimental import pallas as pl
from jax.experimental.pallas import tpu as pltpu
from jax.experimental.pallas import tpu_sc as plsc

_GATHER_WINDOW = 128


def _sc_gather(logits, state2d):
    b = state2d.shape[1]
    a = logits.shape[1]
    mesh = plsc.VectorSubcoreMesh(core_axis_name="c", subcore_axis_name="s")

    @pl.kernel(
        out_type=jax.ShapeDtypeStruct((b, a), logits.dtype),
        mesh=mesh,
        compiler_params=pltpu.CompilerParams(use_tc_tiling_on_sc=False),
    )
    def gather_kernel(x_hbm, i_hbm, o_hbm):
        def body(i_vmem, o_vmem):
            pltpu.sync_copy(x_hbm.at[i_vmem.at[0]], o_vmem)

        pltpu.emit_pipeline(
            body,
            grid=(b // _GATHER_WINDOW,),
            in_specs=[pl.BlockSpec((1, _GATHER_WINDOW), lambda i: (0, i))],
            out_specs=[pl.BlockSpec((_GATHER_WINDOW, a), lambda i: (i, 0))],
            core_axis_name=("c", "s"),
            dimension_semantics=(pltpu.PARALLEL,),
        )(i_hbm, o_hbm)

    return gather_kernel(logits, state2d)


def _tc_softmax(x):
    blk = 2048

    def body(x_ref, o_ref):
        v = x_ref[...]
        m = jnp.max(v, axis=-1, keepdims=True)
        e = jnp.exp(v - m)
        o_ref[...] = e / jnp.sum(e, axis=-1, keepdims=True)

    return pl.pallas_call(
        body,
        out_shape=jax.ShapeDtypeStruct(x.shape, x.dtype),
        grid=(x.shape[0] // blk,),
        in_specs=[pl.BlockSpec((blk, x.shape[1]), lambda i: (i, 0))],
        out_specs=pl.BlockSpec((blk, x.shape[1]), lambda i: (i, 0)),
    )(x)


@jax.jit
def kernel(state, logits):
    state2d = state.astype(jnp.int32).reshape(1, -1)
    gathered = _sc_gather(logits, state2d)
    return _tc_softmax(gathered)
